# baseline (speedup 1.0000x reference)
import functools

import jax
import jax.numpy as jnp
from jax.experimental import pallas as pl
from jax.experimental.pallas import tpu as pltpu

Q = 1024
K = 100000
D = 128
EPS = 1e-8
KBLK = 2048


def _scores_body(q_ref, k_ref, qn_ref, kn_ref, out_ref):
    q = q_ref[...]
    k = k_ref[...]
    dot = jax.lax.dot_general(q, k, (((1,), (1,)), ((), ())))
    denom = jnp.maximum(qn_ref[...] * kn_ref[0:1, :], EPS)
    out_ref[...] = dot / denom


@jax.jit
def _scores(queries, keys):
    qn = jnp.linalg.norm(queries, axis=1, keepdims=True)
    kn = jnp.linalg.norm(keys, axis=1, keepdims=True)
    kn2d = jnp.broadcast_to(kn.T, (8, K))
    grid = (pl.cdiv(K, KBLK),)
    return pl.pallas_call(
        _scores_body,
        grid=grid,
        in_specs=[
            pl.BlockSpec((Q, D), lambda i: (0, 0)),
            pl.BlockSpec((KBLK, D), lambda i: (i, 0)),
            pl.BlockSpec((Q, 1), lambda i: (0, 0)),
            pl.BlockSpec((8, KBLK), lambda i: (0, i)),
        ],
        out_specs=pl.BlockSpec((Q, KBLK), lambda i: (0, i)),
        out_shape=jax.ShapeDtypeStruct((Q, K), jnp.float32),
    )(queries, keys, qn, kn2d)


def kernel(queries, keys):
    scores = _scores(queries, keys)
    order = jnp.argsort(-scores, axis=1)
    sorted_scores = jnp.take_along_axis(scores, order, axis=1)
    return sorted_scores, order


# speedup vs baseline: 2.0594x; 2.0594x over previous
import functools

import jax
import jax.numpy as jnp
from jax import lax
from jax.experimental import pallas as pl
from jax.experimental.pallas import tpu as pltpu
from jax.experimental.pallas import tpu_sc as plsc

Q = 1024
K = 100000
D = 128
EPS = 1e-8
KBLK = 2048

NT = 16
NC = 2
C = 6256
KP = NT * C
NV = C // 16
RADIX = 2048
SHIFTS = (0, 11, 22)
ROWS_PER_CORE = Q // NC

_SIGN_BITS = 0x80000000
_PAD_KEY = -1


def _scores_body(q_ref, k_ref, qn_ref, kn_ref, out_ref):
    dot = jax.lax.dot_general(q_ref[...], k_ref[...], (((1,), (1,)), ((), ())))
    denom = jnp.maximum(qn_ref[...] * kn_ref[0:1, :], EPS)
    out_ref[...] = dot / denom


def _scores(queries, keys):
    qn = jnp.linalg.norm(queries, axis=1, keepdims=True)
    kn = jnp.linalg.norm(keys, axis=1, keepdims=True)
    kn2d = jnp.broadcast_to(kn.T, (8, K))
    return pl.pallas_call(
        _scores_body,
        grid=(pl.cdiv(K, KBLK),),
        in_specs=[
            pl.BlockSpec((Q, D), lambda i: (0, 0)),
            pl.BlockSpec((KBLK, D), lambda i: (i, 0)),
            pl.BlockSpec((Q, 1), lambda i: (0, 0)),
            pl.BlockSpec((8, KBLK), lambda i: (0, i)),
        ],
        out_specs=pl.BlockSpec((Q, KBLK), lambda i: (0, i)),
        out_shape=jax.ShapeDtypeStruct((Q, KP), jnp.float32),
    )(queries, keys, qn, kn2d)


_mesh = plsc.VectorSubcoreMesh(core_axis_name="c", subcore_axis_name="s")


@functools.partial(
    pl.kernel,
    out_type=[
        jax.ShapeDtypeStruct((Q * K,), jnp.float32),
        jax.ShapeDtypeStruct((Q * K,), jnp.int32),
    ],
    mesh=_mesh,
    compiler_params=pltpu.CompilerParams(needs_layout_passes=False),
    scratch_types=[
        pltpu.VMEM_SHARED((KP,), jnp.int32),
        pltpu.VMEM_SHARED((KP,), jnp.int32),
        pltpu.VMEM_SHARED((KP,), jnp.int32),
        pltpu.VMEM_SHARED((KP,), jnp.int32),
        pltpu.VMEM_SHARED((NT * RADIX,), jnp.int32),
        pltpu.VMEM_SHARED((NT * RADIX,), jnp.int32),
        pltpu.VMEM_SHARED((NT * 16,), jnp.int32),
        pltpu.VMEM((C,), jnp.int32),
        pltpu.VMEM((C,), jnp.int32),
        pltpu.VMEM((C,), jnp.int32),
        pltpu.VMEM((C,), jnp.float32),
        pltpu.VMEM((RADIX,), jnp.int32),
        pltpu.VMEM((RADIX,), jnp.int32),
        pltpu.VMEM((NT * 128,), jnp.int32),
        pltpu.VMEM((NT * 128,), jnp.int32),
        pltpu.VMEM((128,), jnp.int32),
        pltpu.VMEM((NT * 16,), jnp.int32),
        pltpu.VMEM((16,), jnp.int32),
        pltpu.SemaphoreType.DMA,
        pltpu.SemaphoreType.DMA,
    ],
)
def _sort_kernel(scores_hbm, sc_out, ord_out, KA, VA, KB, VB, HG, CURG, TS2,
                 keych, valch, posbuf, scorech, cur, hist, A, CURbuf, lexcl,
                 TSl, TSbuf, sem1, sem2):
    c = lax.axis_index("c")
    s = lax.axis_index("s")
    lanes = lax.iota(jnp.int32, 16)
    zeros16 = jnp.zeros((16,), jnp.int32)

    def do_pass(shift, dst_k, dst_v):
        def z(i, _):
            hist[pl.ds(i * 16, 16)] = zeros16
            return 0
        lax.fori_loop(0, RADIX // 16, z, 0)

        def hsweep(i, _):
            kk = plsc.bitcast(keych[pl.ds(i * 16, 16)], jnp.uint32)
            d = ((kk >> shift) & jnp.uint32(RADIX - 1)).astype(jnp.int32)
            cnt, lastm = plsc.scan_count(d)
            old = plsc.load_gather(hist, [d])
            plsc.store_scatter(hist, [d], old + cnt, mask=lastm)
            return 0
        lax.fori_loop(0, NV, hsweep, 0)

        pltpu.sync_copy(hist, HG.at[pl.ds(s * RADIX, RADIX)])
        plsc.subcore_barrier()

        def rd(u, _):
            pltpu.sync_copy(HG.at[pl.ds(u * RADIX + s * 128, 128)],
                            A.at[pl.ds(u * 128, 128)])
            return 0
        lax.fori_loop(0, NT, rd, 0)

        def grp(g, _):
            def inner(u, acc):
                CURbuf[pl.ds(u * 128 + g * 16, 16)] = acc
                return acc + A[pl.ds(u * 128 + g * 16, 16)]
            tot = lax.fori_loop(0, NT, inner, zeros16)
            lexcl[pl.ds(g * 16, 16)] = tot
            return 0
        lax.fori_loop(0, 8, grp, 0)

        def lscan(g, carry):
            v = lexcl[pl.ds(g * 16, 16)]
            inc = plsc.cumsum(v)
            lexcl[pl.ds(g * 16, 16)] = inc - v + carry
            return carry + jnp.sum(v)
        s_t = lax.fori_loop(0, 8, lscan, jnp.int32(0))

        TSbuf[...] = jnp.full((16,), 1, jnp.int32) * s_t
        pltpu.sync_copy(TSbuf, TS2.at[pl.ds(s * 16, 16)])
        plsc.subcore_barrier()
        pltpu.sync_copy(TS2, TSl)
        svec = plsc.load_gather(TSl, [lanes * 16])
        base_t = jnp.sum(jnp.where(lanes < s, svec, 0))

        def fin(g, _):
            gb = lexcl[pl.ds(g * 16, 16)] + base_t
            def inner2(u, _):
                off = u * 128 + g * 16
                CURbuf[pl.ds(off, 16)] = CURbuf[pl.ds(off, 16)] + gb
                return 0
            lax.fori_loop(0, NT, inner2, 0)
            return 0
        lax.fori_loop(0, 8, fin, 0)

        def wr(u, _):
            pltpu.sync_copy(CURbuf.at[pl.ds(u * 128, 128)],
                            CURG.at[pl.ds(u * RADIX + s * 128, 128)])
            return 0
        lax.fori_loop(0, NT, wr, 0)
        plsc.subcore_barrier()

        pltpu.sync_copy(CURG.at[pl.ds(s * RADIX, RADIX)], cur)

        def ssweep(i, _):
            kk = plsc.bitcast(keych[pl.ds(i * 16, 16)], jnp.uint32)
            d = ((kk >> shift) & jnp.uint32(RADIX - 1)).astype(jnp.int32)
            cnt, lastm = plsc.scan_count(d)
            cc = plsc.load_gather(cur, [d])
            posbuf[pl.ds(i * 16, 16)] = cc + cnt - 1
            plsc.store_scatter(cur, [d], cc + cnt, mask=lastm)
            return 0
        lax.fori_loop(0, NV, ssweep, 0)

        cp1 = pltpu.async_copy(keych, dst_k.at[posbuf], sem1)
        cp2 = pltpu.async_copy(valch, dst_v.at[posbuf], sem2)
        cp1.wait()
        cp2.wait()
        plsc.subcore_barrier()

    def do_row(r, _):
        row = c * ROWS_PER_CORE + r
        in_base = row * KP + s * C
        pltpu.sync_copy(scores_hbm.at[pl.ds(in_base, C)], scorech)

        def xform(i, _):
            b = plsc.bitcast(scorech[pl.ds(i * 16, 16)], jnp.uint32)
            asc = jnp.where(b >= jnp.uint32(_SIGN_BITS), ~b, b | jnp.uint32(_SIGN_BITS))
            keych[pl.ds(i * 16, 16)] = plsc.bitcast(~asc, jnp.int32)
            valch[pl.ds(i * 16, 16)] = s * C + i * 16 + lanes
            return 0
        lax.fori_loop(0, NV, xform, 0)

        @pl.when(s == NT - 1)
        def _():
            for j in range(6):
                keych[pl.ds(6160 + j * 16, 16)] = jnp.full((16,), 1, jnp.int32) * jnp.int32(_PAD_KEY)

        do_pass(SHIFTS[0], KA, VA)
        pltpu.sync_copy(KA.at[pl.ds(s * C, C)], keych)
        pltpu.sync_copy(VA.at[pl.ds(s * C, C)], valch)
        do_pass(SHIFTS[1], KB, VB)
        pltpu.sync_copy(KB.at[pl.ds(s * C, C)], keych)
        pltpu.sync_copy(VB.at[pl.ds(s * C, C)], valch)
        do_pass(SHIFTS[2], KA, VA)

        pltpu.sync_copy(KA.at[pl.ds(s * C, C)], keych)
        pltpu.sync_copy(VA.at[pl.ds(s * C, C)], valch)

        def inv(i, _):
            kk = plsc.bitcast(keych[pl.ds(i * 16, 16)], jnp.uint32)
            asc = ~kk
            b = jnp.where(asc >= jnp.uint32(_SIGN_BITS), asc & jnp.uint32(0x7FFFFFFF), ~asc)
            scorech[pl.ds(i * 16, 16)] = plsc.bitcast(b, jnp.float32)
            return 0
        lax.fori_loop(0, NV, inv, 0)

        out_base = row * K + s * C

        @pl.when(s < NT - 1)
        def _():
            pltpu.sync_copy(scorech, sc_out.at[pl.ds(out_base, C)])
            pltpu.sync_copy(valch, ord_out.at[pl.ds(out_base, C)])

        @pl.when(s == NT - 1)
        def _():
            pltpu.sync_copy(scorech.at[pl.ds(0, 6160)],
                            sc_out.at[pl.ds(out_base, 6160)])
            pltpu.sync_copy(valch.at[pl.ds(0, 6160)],
                            ord_out.at[pl.ds(out_base, 6160)])

        plsc.subcore_barrier()
        return 0

    lax.fori_loop(0, ROWS_PER_CORE, do_row, 0)


@jax.jit
def kernel(queries, keys):
    scores = _scores(queries, keys)
    sorted_scores, order = _sort_kernel(scores.reshape(-1))
    return sorted_scores.reshape(Q, K), order.reshape(Q, K)
